# baseline (speedup 1.0000x reference)
import functools

import jax
import jax.numpy as jnp
from jax import lax
from jax.experimental import pallas as pl
from jax.experimental.pallas import tpu as pltpu
from jax.experimental.pallas import tpu_sc as plsc

_NNZ = 268435
_NW = 32
_C = 8384
_MAIN = _NW * _C
_TAIL = _NNZ - _MAIN
_TAIL_PAD = 160
_NIDX = 2 * _NNZ
_CI = 16776
_IMAIN = _NW * _CI
_ITAIL = _NIDX - _IMAIN
_SCALE = float(1.0 / 0.9)
_LANES = 16


def _drop(x, r):
    keep = (r + jnp.float32(0.9)) >= jnp.float32(1.0)
    return jnp.where(keep, x * jnp.float32(_SCALE), jnp.float32(0.0))


def _dropout_body(idx_hbm, vals_hbm, rand_hbm, oidx_hbm, out_hbm,
                  v_v, r_v, o_v, i_v, tv_v, tr_v, ti_v, sem, isem, tsem):
    wid = lax.axis_index("s") * 2 + lax.axis_index("c")
    base = wid * _C
    ibase = wid * _CI

    hi_in = pltpu.async_copy(idx_hbm.at[pl.ds(ibase, _CI)], i_v, isem)
    h1 = pltpu.async_copy(vals_hbm.at[pl.ds(base, _C)], v_v, sem)
    h2 = pltpu.async_copy(rand_hbm.at[pl.ds(base, _C)], r_v, sem)

    @pl.when(wid == 0)
    def _tail_in():
        pltpu.async_copy(vals_hbm.at[pl.ds(_MAIN, _TAIL)],
                         tv_v.at[pl.ds(0, _TAIL)], tsem)
        pltpu.async_copy(rand_hbm.at[pl.ds(_MAIN, _TAIL)],
                         tr_v.at[pl.ds(0, _TAIL)], tsem)

    @pl.when(wid == 1)
    def _tail_idx_in():
        pltpu.async_copy(idx_hbm.at[pl.ds(_IMAIN, _ITAIL)],
                         ti_v.at[pl.ds(0, _ITAIL)], tsem)

    hi_in.wait()
    hi_out = pltpu.async_copy(i_v, oidx_hbm.at[pl.ds(ibase, _CI)], isem)
    h1.wait()
    h2.wait()

    @plsc.parallel_loop(0, _C, _LANES, unroll=4)
    def _main(o):
        o_v[pl.ds(o, _LANES)] = _drop(v_v[pl.ds(o, _LANES)],
                                      r_v[pl.ds(o, _LANES)])

    h3 = pltpu.async_copy(o_v, out_hbm.at[pl.ds(base, _C)], sem)

    @pl.when(wid == 0)
    def _tail_compute():
        pltpu.make_async_copy(vals_hbm.at[pl.ds(_MAIN, _TAIL)],
                              tv_v.at[pl.ds(0, _TAIL)], tsem).wait()
        pltpu.make_async_copy(rand_hbm.at[pl.ds(_MAIN, _TAIL)],
                              tr_v.at[pl.ds(0, _TAIL)], tsem).wait()
        for j in range(_TAIL_PAD // _LANES):
            o = j * _LANES
            tv_v[pl.ds(o, _LANES)] = _drop(tv_v[pl.ds(o, _LANES)],
                                           tr_v[pl.ds(o, _LANES)])
        pltpu.sync_copy(tv_v.at[pl.ds(0, _TAIL)],
                        out_hbm.at[pl.ds(_MAIN, _TAIL)])

    @pl.when(wid == 1)
    def _tail_idx_out():
        pltpu.make_async_copy(idx_hbm.at[pl.ds(_IMAIN, _ITAIL)],
                              ti_v.at[pl.ds(0, _ITAIL)], tsem).wait()
        pltpu.sync_copy(ti_v.at[pl.ds(0, _ITAIL)],
                        oidx_hbm.at[pl.ds(_IMAIN, _ITAIL)])

    hi_out.wait()
    h3.wait()


_dropout_sc = functools.partial(
    pl.kernel,
    out_type=(jax.ShapeDtypeStruct((_NIDX,), jnp.int32),
              jax.ShapeDtypeStruct((_NNZ,), jnp.float32)),
    mesh=plsc.VectorSubcoreMesh(core_axis_name="c", subcore_axis_name="s"),
    scratch_types=[
        pltpu.VMEM((_C,), jnp.float32),
        pltpu.VMEM((_C,), jnp.float32),
        pltpu.VMEM((_C,), jnp.float32),
        pltpu.VMEM((_CI,), jnp.int32),
        pltpu.VMEM((_TAIL_PAD,), jnp.float32),
        pltpu.VMEM((_TAIL_PAD,), jnp.float32),
        pltpu.VMEM((48,), jnp.int32),
        pltpu.SemaphoreType.DMA,
        pltpu.SemaphoreType.DMA,
        pltpu.SemaphoreType.DMA,
    ],
)(_dropout_body)


def kernel(x_indices, x_values, rand_vals):
    out_idx, out_values = _dropout_sc(x_indices.reshape(_NIDX),
                                      x_values, rand_vals)
    return out_idx.reshape(2, _NNZ), out_values


# speedup vs baseline: 1.2194x; 1.2194x over previous
import functools

import jax
import jax.numpy as jnp
from jax import lax
from jax.experimental import pallas as pl
from jax.experimental.pallas import tpu as pltpu
from jax.experimental.pallas import tpu_sc as plsc

_NNZ = 268435
_NW = 32
_C = 8384
_H = _C // 2
_MAIN = _NW * _C
_TAIL = _NNZ - _MAIN
_TAIL_PAD = 160
_SCALE = float(1.0 / 0.9)
_LANES = 16


def _drop(x, r):
    keep = (r + jnp.float32(0.9)) >= jnp.float32(1.0)
    return jnp.where(keep, x * jnp.float32(_SCALE), jnp.float32(0.0))


def _dropout_body(vals_hbm, rand_hbm, out_hbm,
                  v_v, r_v, o_v, tv_v, tr_v, sem, sem_b, tsem):
    wid = lax.axis_index("s") * 2 + lax.axis_index("c")
    base = wid * _C
    h1a = pltpu.async_copy(vals_hbm.at[pl.ds(base, _H)],
                           v_v.at[pl.ds(0, _H)], sem)
    h2a = pltpu.async_copy(rand_hbm.at[pl.ds(base, _H)],
                           r_v.at[pl.ds(0, _H)], sem)
    h1b = pltpu.async_copy(vals_hbm.at[pl.ds(base + _H, _H)],
                           v_v.at[pl.ds(_H, _H)], sem_b)
    h2b = pltpu.async_copy(rand_hbm.at[pl.ds(base + _H, _H)],
                           r_v.at[pl.ds(_H, _H)], sem_b)

    is_tail_worker = wid == 0

    @pl.when(is_tail_worker)
    def _tail_in():
        pltpu.async_copy(vals_hbm.at[pl.ds(_MAIN, _TAIL)],
                         tv_v.at[pl.ds(0, _TAIL)], tsem)
        pltpu.async_copy(rand_hbm.at[pl.ds(_MAIN, _TAIL)],
                         tr_v.at[pl.ds(0, _TAIL)], tsem)

    h1a.wait()
    h2a.wait()

    @plsc.parallel_loop(0, _H, _LANES, unroll=4)
    def _main_a(o):
        o_v[pl.ds(o, _LANES)] = _drop(v_v[pl.ds(o, _LANES)],
                                      r_v[pl.ds(o, _LANES)])

    h3a = pltpu.async_copy(o_v.at[pl.ds(0, _H)],
                           out_hbm.at[pl.ds(base, _H)], sem)

    h1b.wait()
    h2b.wait()

    @plsc.parallel_loop(_H, _C, _LANES, unroll=4)
    def _main_b(o):
        o_v[pl.ds(o, _LANES)] = _drop(v_v[pl.ds(o, _LANES)],
                                      r_v[pl.ds(o, _LANES)])

    h3b = pltpu.async_copy(o_v.at[pl.ds(_H, _H)],
                           out_hbm.at[pl.ds(base + _H, _H)], sem_b)

    @pl.when(is_tail_worker)
    def _tail_compute():
        pltpu.make_async_copy(vals_hbm.at[pl.ds(_MAIN, _TAIL)],
                              tv_v.at[pl.ds(0, _TAIL)], tsem).wait()
        pltpu.make_async_copy(rand_hbm.at[pl.ds(_MAIN, _TAIL)],
                              tr_v.at[pl.ds(0, _TAIL)], tsem).wait()
        for j in range(_TAIL_PAD // _LANES):
            o = j * _LANES
            tv_v[pl.ds(o, _LANES)] = _drop(tv_v[pl.ds(o, _LANES)],
                                           tr_v[pl.ds(o, _LANES)])
        pltpu.sync_copy(tv_v.at[pl.ds(0, _TAIL)],
                        out_hbm.at[pl.ds(_MAIN, _TAIL)])

    h3a.wait()
    h3b.wait()


_dropout_sc = functools.partial(
    pl.kernel,
    out_type=jax.ShapeDtypeStruct((_NNZ,), jnp.float32),
    mesh=plsc.VectorSubcoreMesh(core_axis_name="c", subcore_axis_name="s"),
    scratch_types=[
        pltpu.VMEM((_C,), jnp.float32),
        pltpu.VMEM((_C,), jnp.float32),
        pltpu.VMEM((_C,), jnp.float32),
        pltpu.VMEM((_TAIL_PAD,), jnp.float32),
        pltpu.VMEM((_TAIL_PAD,), jnp.float32),
        pltpu.SemaphoreType.DMA,
        pltpu.SemaphoreType.DMA,
        pltpu.SemaphoreType.DMA,
    ],
)(_dropout_body)


def kernel(x_indices, x_values, rand_vals):
    out_values = _dropout_sc(x_values, rand_vals)
    return x_indices, out_values
